# baseline (speedup 1.0000x reference)
import functools

import jax
import jax.numpy as jnp
from jax import lax
from jax.experimental import pallas as pl
from jax.experimental.pallas import tpu as pltpu
from jax.experimental.pallas import tpu_sc as plsc

DIM = 4096
NUM_E = 64
TOPK = 8
NT = 16384
NSEL = NT * TOPK
NW = 32
CHUNK = NSEL // NW
NVEC = CHUNK // 16
NROW = CHUNK // 128
BT = 1024


def _top8(p):
    iota = lax.broadcasted_iota(jnp.int32, p.shape, 1)
    key = (lax.bitcast_convert_type(p, jnp.int32) & ~63) | (63 - iota)
    vals = []
    idxs = []
    acc = jnp.zeros(p.shape, jnp.int32)
    for _ in range(TOPK):
        mk = jnp.max(key, axis=1, keepdims=True)
        jv = 63 - (mk & 63)
        idxs.append(jv)
        vals.append(lax.bitcast_convert_type((mk & ~63) | 32, jnp.float32))
        acc = acc + (jv == iota).astype(jnp.int32)
        key = jnp.where(key == mk, 0, key)
    return (jnp.concatenate(vals, axis=1), jnp.concatenate(idxs, axis=1), acc)


def _router_tc_body(x_ref, w_ref, vals_ref, idx_ref, hist_ref):
    s = lax.dot_general(
        x_ref[...], w_ref[...],
        dimension_numbers=(((1,), (1,)), ((), ())),
        preferred_element_type=jnp.float32,
    )
    m = jnp.max(s, axis=1, keepdims=True)
    e = jnp.exp(s - m)
    p = e / jnp.sum(e, axis=1, keepdims=True)
    v, i, acc = _top8(p)
    vals_ref[...] = v
    idx_ref[...] = i
    rows = [jnp.sum(acc[h * 512:(h + 1) * 512, :], axis=0, keepdims=True)
            for h in range(BT // 512)]
    hist_ref[...] = jnp.concatenate(rows, axis=0)[None]


def _router_tc(x, gate_weight):
    return pl.pallas_call(
        _router_tc_body,
        grid=(NT // BT,),
        in_specs=[
            pl.BlockSpec((BT, DIM), lambda i: (i, 0)),
            pl.BlockSpec((NUM_E, DIM), lambda i: (0, 0)),
        ],
        out_specs=[
            pl.BlockSpec((BT, TOPK), lambda i: (i, 0)),
            pl.BlockSpec((BT, TOPK), lambda i: (i, 0)),
            pl.BlockSpec((1, BT // 512, NUM_E), lambda i: (i, 0, 0)),
        ],
        out_shape=[
            jax.ShapeDtypeStruct((NT, TOPK), jnp.float32),
            jax.ShapeDtypeStruct((NT, TOPK), jnp.int32),
            jax.ShapeDtypeStruct((NT // BT, BT // 512, NUM_E), jnp.int32),
        ],
    )(x, gate_weight)


def _scan_base():
    rc0, _ = plsc.scan_count(jnp.zeros((16,), jnp.int32))
    return jnp.min(rc0)


_MESH1 = plsc.VectorSubcoreMesh(
    core_axis_name="c", subcore_axis_name="s", num_cores=1)
NW2 = 16
CHUNK2 = NSEL // NW2
NVEC2 = CHUNK2 // 16
NROW2 = CHUNK2 // 128


@functools.partial(
    pl.kernel,
    out_type=(
        jax.ShapeDtypeStruct((NSEL,), jnp.float32),
        jax.ShapeDtypeStruct((NSEL,), jnp.int32),
        jax.ShapeDtypeStruct((NUM_E,), jnp.int32),
    ),
    mesh=_MESH1,
    compiler_params=pltpu.CompilerParams(needs_layout_passes=False),
    scratch_types=[
        pltpu.VMEM((CHUNK2,), jnp.int32),
        pltpu.VMEM((NROW2, 128), jnp.float32),
        pltpu.VMEM((NROW2, 128), jnp.int32),
        pltpu.VMEM((NROW2, 128), jnp.int32),
        pltpu.VMEM((NW, NUM_E), jnp.int32),
        pltpu.VMEM((NUM_E,), jnp.int32),
        pltpu.VMEM((NUM_E,), jnp.int32),
        pltpu.VMEM_SHARED((NSEL,), jnp.float32),
        pltpu.VMEM_SHARED((NSEL,), jnp.int32),
        pltpu.SemaphoreType.DMA,
    ],
)
def _scatter_sc(sel_hbm, scores_hbm, hist_hbm, out_s_hbm, out_t_hbm, cnt_hbm,
                sel_v, sc_v, pos_v, tok_v, hist_v, cnt_v, tot_v,
                sh_s, sh_t, sem):
    w = lax.axis_index("s")
    pltpu.sync_copy(sel_hbm.at[pl.ds(w * CHUNK2, CHUNK2)], sel_v)
    pltpu.sync_copy(scores_hbm.at[pl.ds(w * NROW2, NROW2)], sc_v)
    pltpu.sync_copy(hist_hbm, hist_v)
    base = _scan_base()

    carry = jnp.int32(0)
    for c in range(NUM_E // 16):
        tot = jnp.zeros((16,), jnp.int32)
        part = jnp.zeros((16,), jnp.int32)
        for r in range(NW):
            h = hist_v[r, pl.ds(c * 16, 16)]
            tot = tot + h
            part = part + jnp.where(r < 2 * w, h, 0)
        excl = plsc.cumsum(tot) - tot
        cnt_v[pl.ds(c * 16, 16)] = excl + carry + part
        tot_v[pl.ds(c * 16, 16)] = tot
        carry = carry + jnp.sum(tot)

    @pl.when(w == 0)
    def _():
        pltpu.sync_copy(tot_v, cnt_hbm)

    lane = lax.iota(jnp.int32, 16)
    first = w * CHUNK2

    def tbody(j, carryv):
        tok_v[j // 8, pl.ds((j % 8) * 16, 16)] = lax.shift_right_logical(
            first + j * 16 + lane, 3)
        return carryv

    lax.fori_loop(0, NVEC2, tbody, 0)

    copies = []
    for r in range(NROW2):
        def body(jj, carryv, r=r):
            j = r * 8 + jj
            v = sel_v[pl.ds(j * 16, 16)]
            rc, last = plsc.scan_count(v)
            cb = plsc.load_gather(cnt_v, [v])
            pos_v[r, pl.ds(jj * 16, 16)] = cb + rc - base
            plsc.addupdate_scatter(cnt_v, [v], rc + (1 - base), mask=last)
            return carryv

        lax.fori_loop(0, 8, body, 0)
        copies.append(
            pltpu.async_copy(sc_v.at[r], sh_s.at[pos_v.at[r]], sem))
        copies.append(
            pltpu.async_copy(tok_v.at[r], sh_t.at[pos_v.at[r]], sem))
    for cp in copies:
        cp.wait()
    plsc.subcore_barrier()
    pltpu.sync_copy(sh_s.at[pl.ds(w * CHUNK2, CHUNK2)],
                    out_s_hbm.at[pl.ds(w * CHUNK2, CHUNK2)])
    pltpu.sync_copy(sh_t.at[pl.ds(w * CHUNK2, CHUNK2)],
                    out_t_hbm.at[pl.ds(w * CHUNK2, CHUNK2)])


def kernel(x, gate_weight):
    top_vals, top_idx, hist = _router_tc(x, gate_weight)
    sel_flat = top_idx.reshape(-1)
    scores_2d = top_vals.reshape(NSEL // 128, 128)
    out_scores, out_tok, counts = _scatter_sc(
        sel_flat, scores_2d, hist.reshape(NW, NUM_E))
    return (out_scores, out_tok, counts)


# speedup vs baseline: 1.0312x; 1.0312x over previous
import functools

import jax
import jax.numpy as jnp
from jax import lax
from jax.experimental import pallas as pl
from jax.experimental.pallas import tpu as pltpu
from jax.experimental.pallas import tpu_sc as plsc

DIM = 4096
NUM_E = 64
TOPK = 8
NT = 16384
NSEL = NT * TOPK
NW = 32
CHUNK = NSEL // NW
NVEC = CHUNK // 16
NROW = CHUNK // 128
BT = 1024


def _top8(p):
    iota = lax.broadcasted_iota(jnp.int32, p.shape, 1)
    key = (lax.bitcast_convert_type(p, jnp.int32) & ~63) | (63 - iota)
    vals = []
    idxs = []
    acc = jnp.zeros(p.shape, jnp.int32)
    for _ in range(TOPK):
        mk = jnp.max(key, axis=1, keepdims=True)
        jv = 63 - (mk & 63)
        idxs.append(jv)
        vals.append(lax.bitcast_convert_type((mk & ~63) | 32, jnp.float32))
        acc = acc + (jv == iota).astype(jnp.int32)
        key = jnp.where(key == mk, 0, key)
    return (jnp.concatenate(vals, axis=1), jnp.concatenate(idxs, axis=1), acc)


def _router_tc_body(x_ref, w_ref, vals_ref, idx_ref, hist_ref):
    s = lax.dot_general(
        x_ref[...], w_ref[...],
        dimension_numbers=(((1,), (1,)), ((), ())),
        preferred_element_type=jnp.float32,
    )
    m = jnp.max(s, axis=1, keepdims=True)
    e = jnp.exp(s - m)
    p = e / jnp.sum(e, axis=1, keepdims=True)
    v, i, acc = _top8(p)
    vals_ref[...] = v
    idx_ref[...] = i
    rows = [jnp.sum(acc[h * 512:(h + 1) * 512, :], axis=0, keepdims=True)
            for h in range(BT // 512)]
    hist_ref[...] = jnp.concatenate(rows, axis=0)[None]


def _router_tc(x, gate_weight):
    return pl.pallas_call(
        _router_tc_body,
        grid=(NT // BT,),
        in_specs=[
            pl.BlockSpec((BT, DIM), lambda i: (i, 0)),
            pl.BlockSpec((NUM_E, DIM), lambda i: (0, 0)),
        ],
        out_specs=[
            pl.BlockSpec((BT, TOPK), lambda i: (i, 0)),
            pl.BlockSpec((BT, TOPK), lambda i: (i, 0)),
            pl.BlockSpec((1, BT // 512, NUM_E), lambda i: (i, 0, 0)),
        ],
        out_shape=[
            jax.ShapeDtypeStruct((NT, TOPK), jnp.float32),
            jax.ShapeDtypeStruct((NT, TOPK), jnp.int32),
            jax.ShapeDtypeStruct((NT // BT, BT // 512, NUM_E), jnp.int32),
        ],
    )(x, gate_weight)


def _scan_base():
    rc0, _ = plsc.scan_count(jnp.zeros((16,), jnp.int32))
    return jnp.min(rc0)


_MESH1 = plsc.VectorSubcoreMesh(
    core_axis_name="c", subcore_axis_name="s", num_cores=1)
NW2 = 16
CHUNK2 = NSEL // NW2
NVEC2 = CHUNK2 // 16
NROW2 = CHUNK2 // 128


@functools.partial(
    pl.kernel,
    out_type=(
        jax.ShapeDtypeStruct((NSEL,), jnp.float32),
        jax.ShapeDtypeStruct((NSEL,), jnp.int32),
        jax.ShapeDtypeStruct((NUM_E,), jnp.int32),
    ),
    mesh=_MESH1,
    compiler_params=pltpu.CompilerParams(needs_layout_passes=False),
    scratch_types=[
        pltpu.VMEM((CHUNK2,), jnp.int32),
        pltpu.VMEM((NROW2, 128), jnp.float32),
        pltpu.VMEM((NROW2, 128), jnp.int32),
        pltpu.VMEM((NROW2, 128), jnp.int32),
        pltpu.VMEM((NW, NUM_E), jnp.int32),
        pltpu.VMEM((NUM_E,), jnp.int32),
        pltpu.VMEM((NUM_E,), jnp.int32),
        pltpu.VMEM_SHARED((NSEL,), jnp.float32),
        pltpu.VMEM_SHARED((NSEL,), jnp.int32),
        pltpu.SemaphoreType.DMA,
    ],
)
def _scatter_sc(sel_hbm, scores_hbm, hist_hbm, out_s_hbm, out_t_hbm, cnt_hbm,
                sel_v, sc_v, pos_v, tok_v, hist_v, cnt_v, tot_v,
                sh_s, sh_t, sem):
    w = lax.axis_index("s")
    pltpu.sync_copy(sel_hbm.at[pl.ds(w * CHUNK2, CHUNK2)], sel_v)
    pltpu.sync_copy(scores_hbm.at[pl.ds(w * NROW2, NROW2)], sc_v)
    pltpu.sync_copy(hist_hbm, hist_v)
    base = _scan_base()

    carry = jnp.int32(0)
    for c in range(NUM_E // 16):
        tot = jnp.zeros((16,), jnp.int32)
        part = jnp.zeros((16,), jnp.int32)
        for r in range(NW):
            h = hist_v[r, pl.ds(c * 16, 16)]
            tot = tot + h
            part = part + jnp.where(r < 2 * w, h, 0)
        excl = plsc.cumsum(tot) - tot
        cnt_v[pl.ds(c * 16, 16)] = excl + carry + part
        tot_v[pl.ds(c * 16, 16)] = tot
        carry = carry + jnp.sum(tot)

    @pl.when(w == 0)
    def _():
        pltpu.sync_copy(tot_v, cnt_hbm)

    lane = lax.iota(jnp.int32, 16)
    first = w * CHUNK2

    copies = []
    for r in range(NROW2):
        def body(jj, carryv, r=r):
            j = r * 8 + jj
            v = sel_v[pl.ds(j * 16, 16)]
            rc, last = plsc.scan_count(v)
            cb = plsc.load_gather(cnt_v, [v])
            pos = cb + rc - base
            plsc.addupdate_scatter(cnt_v, [v], rc + (1 - base), mask=last)
            col = jj * 16
            pos_v[r, pl.ds(col, 16)] = pos
            tok_v[r, pl.ds(col, 16)] = lax.shift_right_logical(
                first + j * 16 + lane, 3)
            return carryv

        lax.fori_loop(0, 8, body, 0)
        copies.append(
            pltpu.async_copy(sc_v.at[r], sh_s.at[pos_v.at[r]], sem))
        copies.append(
            pltpu.async_copy(tok_v.at[r], sh_t.at[pos_v.at[r]], sem))
    for cp in copies:
        cp.wait()
    plsc.subcore_barrier()
    pltpu.sync_copy(sh_s.at[pl.ds(w * CHUNK2, CHUNK2)],
                    out_s_hbm.at[pl.ds(w * CHUNK2, CHUNK2)])
    pltpu.sync_copy(sh_t.at[pl.ds(w * CHUNK2, CHUNK2)],
                    out_t_hbm.at[pl.ds(w * CHUNK2, CHUNK2)])


def kernel(x, gate_weight):
    top_vals, top_idx, hist = _router_tc(x, gate_weight)
    sel_flat = top_idx.reshape(-1)
    scores_2d = top_vals.reshape(NSEL // 128, 128)
    out_scores, out_tok, counts = _scatter_sc(
        sel_flat, scores_2d, hist.reshape(NW, NUM_E))
    return (out_scores, out_tok, counts)
